# baseline (speedup 1.0000x reference)
import jax
import jax.numpy as jnp
from jax import lax
from jax.experimental import pallas as pl
from jax.experimental.pallas import tpu as pltpu
from jax.experimental.pallas import tpu_sc as plsc

_LANES = 16


def _sc_body(yt_hbm, yp_hbm, tvec_hbm, out_hbm, idx_v, yt_v, yp_v, out_v, sem):
    is_owner = (lax.axis_index("c") == 0) & (lax.axis_index("s") == 0)

    @pl.when(is_owner)
    def _():
        pltpu.sync_copy(tvec_hbm, idx_v)
        tv = idx_v[...]
        t = lax.reduce_max(tv, axes=(0,))
        base = pl.multiple_of((t // 8) * 8, 8)
        r = t - base
        cp_t = pltpu.make_async_copy(yt_hbm.at[pl.ds(base, 8), :], yt_v, sem)
        cp_p = pltpu.make_async_copy(yp_hbm.at[pl.ds(base, 8), :], yp_v, sem)
        cp_t.start()
        cp_p.start()
        cp_t.wait()
        cp_p.wait()
        d = yp_v[r, pl.ds(0, _LANES)] - yt_v[r, pl.ds(0, _LANES)]
        lane = lax.broadcasted_iota(jnp.int32, (_LANES,), 0)
        val = jnp.where(lane == 0, -(d * d), jnp.zeros((_LANES,), jnp.float32))
        out_v[...] = val
        pltpu.sync_copy(out_v, out_hbm)


def _sc_call(yt_t, yp_t, tvec):
    mesh = plsc.VectorSubcoreMesh(
        core_axis_name="c", subcore_axis_name="s", num_cores=1, num_subcores=1
    )
    return pl.kernel(
        _sc_body,
        out_type=jax.ShapeDtypeStruct((_LANES,), jnp.float32),
        mesh=mesh,
        compiler_params=pltpu.CompilerParams(
            needs_layout_passes=False, use_tc_tiling_on_sc=True
        ),
        scratch_types=[
            pltpu.VMEM((_LANES,), jnp.int32),
            pltpu.VMEM((8, 128), jnp.float32),
            pltpu.VMEM((8, 128), jnp.float32),
            pltpu.VMEM((_LANES,), jnp.float32),
            pltpu.SemaphoreType.DMA,
        ],
    )(yt_t, yp_t, tvec)


def kernel(y_true, y_pred, target):
    t = jnp.asarray(target, jnp.int32)
    tvec = jnp.full((_LANES,), t, dtype=jnp.int32)
    out = _sc_call(y_true.T, y_pred.T, tvec)
    return out[0]


# speedup vs baseline: 1.0828x; 1.0828x over previous
import jax
import jax.numpy as jnp
from jax import lax
from jax.experimental import pallas as pl
from jax.experimental.pallas import tpu as pltpu
from jax.experimental.pallas import tpu_sc as plsc

_LANES = 16


def _sc_body(yt_hbm, yp_hbm, tvec_hbm, out_hbm, idx_v, yt_v, yp_v, out_v, sem):
    is_owner = (lax.axis_index("c") == 0) & (lax.axis_index("s") == 0)

    @pl.when(is_owner)
    def _():
        pltpu.sync_copy(tvec_hbm, idx_v)
        tv = idx_v[...]
        t = lax.reduce_max(tv, axes=(0,))
        base = pl.multiple_of((t // 8) * 8, 8)
        r = t - base
        cp_t = pltpu.make_async_copy(yt_hbm.at[pl.ds(base, 8), :], yt_v, sem)
        cp_p = pltpu.make_async_copy(yp_hbm.at[pl.ds(base, 8), :], yp_v, sem)
        cp_t.start()
        cp_p.start()
        cp_t.wait()
        cp_p.wait()
        d = yp_v[r, pl.ds(0, _LANES)] - yt_v[r, pl.ds(0, _LANES)]
        lane = lax.broadcasted_iota(jnp.int32, (_LANES,), 0)
        val = jnp.where(lane == 0, -(d * d), jnp.zeros((_LANES,), jnp.float32))
        out_v[...] = val
        pltpu.sync_copy(out_v, out_hbm)


def _sc_call(yt_t, yp_t, tvec):
    mesh = plsc.VectorSubcoreMesh(
        core_axis_name="c", subcore_axis_name="s", num_cores=1, num_subcores=1
    )
    return pl.kernel(
        _sc_body,
        out_type=jax.ShapeDtypeStruct((_LANES,), jnp.float32),
        mesh=mesh,
        compiler_params=pltpu.CompilerParams(
            needs_layout_passes=False, use_tc_tiling_on_sc=True,
            skip_device_barrier=True
        ),
        scratch_types=[
            pltpu.VMEM((_LANES,), jnp.int32),
            pltpu.VMEM((8, 128), jnp.float32),
            pltpu.VMEM((8, 128), jnp.float32),
            pltpu.VMEM((_LANES,), jnp.float32),
            pltpu.SemaphoreType.DMA,
        ],
    )(yt_t, yp_t, tvec)


def kernel(y_true, y_pred, target):
    t = jnp.asarray(target, jnp.int32)
    tvec = jnp.full((_LANES,), t, dtype=jnp.int32)
    out = _sc_call(y_true.T, y_pred.T, tvec)
    return out[0]
